# baseline (speedup 1.0000x reference)
import functools

import jax
import jax.numpy as jnp
from jax import lax
from jax.experimental import pallas as pl
from jax.experimental.pallas import tpu as pltpu
from jax.experimental.pallas import tpu_sc as plsc

B = 16384
NUM_DIM = 13
NUM_FIELDS = 26
VOCAB = 100000
EMB_DIM = 16
EPS = 1e-5

NT = 4
FPT = 8
IDX_ROWS = NT * B * FPT // 128

VTILES = VOCAB // 128
VTAIL = VOCAB - VTILES * 128
VOCAB_P = (VTILES + 1) * 128
RPF = VOCAB_P * EMB_DIM // 128
GRP = 11
NGRP = VTILES // GRP
ITEMS = NUM_FIELDS * NGRP

RB = 2048
NBLK = B // RB
H1 = 256
H2 = 128


def _sc_transpose(tt, aux):
  info = plsc.get_sparse_core_info()
  nw = info.num_cores * info.num_subcores
  n_iter = (ITEMS + nw - 1) // nw
  n_rounds = (n_iter + 1) // 2
  cols = GRP * 128
  orows = cols * EMB_DIM // 128
  mesh = plsc.VectorSubcoreMesh(core_axis_name="c", subcore_axis_name="s")

  @functools.partial(
      pl.kernel,
      mesh=mesh,
      compiler_params=pltpu.CompilerParams(use_tc_tiling_on_sc=True,
                                           needs_layout_passes=False),
      out_type=jax.ShapeDtypeStruct((NUM_FIELDS * RPF * 128,), jnp.float32),
      scratch_types=[
          pltpu.VMEM((EMB_DIM, cols), jnp.float32),
          pltpu.VMEM((EMB_DIM, cols), jnp.float32),
          pltpu.VMEM((orows * 128,), jnp.float32),
          pltpu.VMEM((orows * 128,), jnp.float32),
          pltpu.VMEM((1024,), jnp.float32),
          pltpu.SemaphoreType.DMA,
          pltpu.SemaphoreType.DMA,
          pltpu.SemaphoreType.DMA,
          pltpu.SemaphoreType.DMA,
      ],
  )
  def transpose_kernel(tt_hbm, aux_hbm, out_hbm, in0, in1, ob0, ob1, stage1d,
                       ws0, ws1, is0, is1):
    wid = lax.axis_index("s") * info.num_cores + lax.axis_index("c")
    lanes = jax.lax.iota(jnp.int32, 16)
    ins = [in0, in1]
    obs = [ob0, ob1]
    wss = [ws0, ws1]
    iss = [is0, is1]

    pltpu.async_copy(tt_hbm.at[wid // NGRP, :,
                               pl.ds((wid - (wid // NGRP) * NGRP) * cols, cols)],
                     in0, is0)

    def round_body(r, carry):
      for b in range(2):
        i = 2 * r + b
        k = i * nw + wid

        @pl.when(k < ITEMS)
        def _(b=b, k=k):
          in_b, out_b, ws_b = ins[b], obs[b], wss[b]
          f = k // NGRP
          g = k - f * NGRP
          pltpu.make_async_copy(tt_hbm.at[0, :, pl.ds(0, cols)], in_b,
                                iss[b]).wait()
          kn = k + nw

          @pl.when(kn < ITEMS)
          def _prefetch():
            fn = kn // NGRP
            gn = kn - fn * NGRP
            pltpu.async_copy(tt_hbm.at[fn, :, pl.ds(gn * cols, cols)],
                             ins[(b + 1) % 2], iss[(b + 1) % 2])

          @pl.when(r > 0)
          def _drain():
            pltpu.make_async_copy(out_b, out_hbm.at[pl.ds(0, orows * 128)],
                                  ws_b).wait()

          def tile_body(t, carry2):
            for s in range(16):
              perm = (lanes + s) & 15
              pl16 = perm * 16 + lanes
              for j in range(8):
                base = t * 128 + j * 16
                vals = plsc.load_gather(in_b, [lanes, base + perm])
                plsc.store_scatter(out_b, [base * 16 + pl16], vals)
            return carry2

          lax.fori_loop(0, GRP, tile_body, 0)
          pltpu.async_copy(
              out_b, out_hbm.at[pl.ds((f * RPF + g * orows) * 128,
                                      orows * 128)], ws_b)

      return carry

    lax.fori_loop(0, n_rounds, round_body, 0)
    for b in range(2):
      pltpu.make_async_copy(obs[b], out_hbm.at[pl.ds(0, orows * 128)],
                            wss[b]).wait()

    @pl.when(wid < NUM_FIELDS)
    def _tail():
      pltpu.sync_copy(aux_hbm.at[pl.ds(wid * 1024, 1024)], stage1d)
      pltpu.sync_copy(stage1d,
                      out_hbm.at[pl.ds((wid * RPF + VTILES * 16) * 128, 1024)])

  return transpose_kernel(tt, aux)


def _sc_gather(tab_lin, idx_t):
  info = plsc.get_sparse_core_info()
  nw = info.num_cores * info.num_subcores
  bpw = B // nw
  n_chunks = NT * 4
  bpc = 128
  mesh = plsc.VectorSubcoreMesh(core_axis_name="c", subcore_axis_name="s")
  eshape = jax.ShapeDtypeStruct((B, 128), jnp.float32)

  @functools.partial(
      pl.kernel,
      mesh=mesh,
      compiler_params=pltpu.CompilerParams(use_tc_tiling_on_sc=False),
      out_type=[eshape] * NT,
      scratch_types=[
          pltpu.VMEM((128, 128), jnp.int32),
          pltpu.VMEM((FPT * bpc, EMB_DIM), jnp.float32),
          pltpu.VMEM((FPT * bpc, EMB_DIM), jnp.float32),
          pltpu.VMEM((bpc, 128), jnp.float32),
          pltpu.VMEM((bpc, 128), jnp.float32),
          pltpu.SemaphoreType.DMA,
          pltpu.SemaphoreType.DMA,
          pltpu.SemaphoreType.DMA,
          pltpu.SemaphoreType.DMA,
      ],
  )
  def gather_kernel(tab_hbm, idx_hbm, e0, e1, e2, e3, idx_v, ga, gb, ta, tb,
                    gsem_a, gsem_b, wsem_a, wsem_b):
    wid = lax.axis_index("s") * info.num_cores + lax.axis_index("c")
    outs = [e0, e1, e2, e3]
    gbufs = [ga, gb]
    tbufs = [ta, tb]
    gsems = [gsem_a, gsem_b]
    wsems = [wsem_a, wsem_b]

    pltpu.sync_copy(idx_hbm.at[pl.ds(wid * 128, 128)], idx_v)
    zeros16 = jnp.zeros((16,), jnp.float32)

    def live_fields(k):
      return FPT if k // 4 < NT - 1 else NUM_FIELDS - (NT - 1) * FPT

    def fire(k):
      gbuf = gbufs[k % 2]
      gsem = gsems[k % 2]
      return [
          pltpu.async_copy(tab_hbm.at[idx_v.at[k * FPT + j]],
                           gbuf.at[pl.ds(j * bpc, bpc)], gsem)
          for j in range(live_fields(k))
      ]

    pend_g = fire(0)
    pend_w = {}
    for k in range(n_chunks):
      for h in pend_g:
        h.wait()
      if k + 1 < n_chunks:
        pend_g = fire(k + 1)
      if k - 2 in pend_w:
        for h in pend_w.pop(k - 2):
          h.wait()
      gbuf = gbufs[k % 2]
      tbuf = tbufs[k % 2]
      nlive = live_fields(k)

      def repack(db, carry, gbuf=gbuf, tbuf=tbuf, nlive=nlive):
        for v in range(nlive):
          tbuf[db, pl.ds(v * EMB_DIM, EMB_DIM)] = gbuf[v * bpc + db, :]
        for v in range(nlive, FPT):
          tbuf[db, pl.ds(v * EMB_DIM, EMB_DIM)] = zeros16
        return carry

      lax.fori_loop(0, bpc, repack, 0)
      t = k // 4
      c = k - t * 4
      pend_w[k] = [
          pltpu.async_copy(tbuf, outs[t].at[pl.ds(wid * bpw + c * bpc, bpc)],
                           wsems[k % 2])
      ]
    for kk in sorted(pend_w):
      for h in pend_w[kk]:
        h.wait()

  return gather_kernel(tab_lin, idx_t)


def _mlp_body(xn_ref, e0_ref, e1_ref, e2_ref, e3_ref, w1a_ref, w1b0_ref,
              w1b1_ref, w1b2_ref, w1b3_ref, b1_ref, g1_ref, be1_ref, w2_ref,
              b2_ref, g2_ref, be2_ref, wpi_ref, bpi_ref, wmu_ref, bmu_ref,
              pi_ref, mu_ref, h1_s, h2_s, s1, q1, s2, q2):
  p = pl.program_id(0)
  i = pl.program_id(1)
  inv_b = 1.0 / B

  @pl.when(p == 0)
  def _phase0():
    @pl.when(i == 0)
    def _():
      s1[...] = jnp.zeros_like(s1)
      q1[...] = jnp.zeros_like(q1)

    h = (jnp.dot(xn_ref[...], w1a_ref[...], preferred_element_type=jnp.float32)
         + jnp.dot(e0_ref[...], w1b0_ref[...], preferred_element_type=jnp.float32)
         + jnp.dot(e1_ref[...], w1b1_ref[...], preferred_element_type=jnp.float32)
         + jnp.dot(e2_ref[...], w1b2_ref[...], preferred_element_type=jnp.float32)
         + jnp.dot(e3_ref[...], w1b3_ref[...], preferred_element_type=jnp.float32)
         + b1_ref[...])
    h1_s[pl.ds(i * RB, RB), :] = h
    s1[...] += jnp.sum(h, axis=0, keepdims=True)
    q1[...] += jnp.sum(h * h, axis=0, keepdims=True)

  @pl.when(p == 1)
  def _phase1():
    @pl.when(i == 0)
    def _():
      s2[...] = jnp.zeros_like(s2)
      q2[...] = jnp.zeros_like(q2)

    m = s1[...] * inv_b
    v = q1[...] * inv_b - m * m
    a = g1_ref[...] * lax.rsqrt(v + EPS)
    c = be1_ref[...] - m * a
    h = h1_s[pl.ds(i * RB, RB), :]
    hn = jnp.maximum(h * a + c, 0.0)
    h2 = jnp.dot(hn, w2_ref[...], preferred_element_type=jnp.float32) + b2_ref[...]
    h2_s[pl.ds(i * RB, RB), :] = h2
    s2[...] += jnp.sum(h2, axis=0, keepdims=True)
    q2[...] += jnp.sum(h2 * h2, axis=0, keepdims=True)

  @pl.when(p == 2)
  def _phase2():
    m = s2[...] * inv_b
    v = q2[...] * inv_b - m * m
    a = g2_ref[...] * lax.rsqrt(v + EPS)
    c = be2_ref[...] - m * a
    h = h2_s[pl.ds(i * RB, RB), :]
    hn = jnp.maximum(h * a + c, 0.0)
    logit = jnp.dot(hn, wpi_ref[...], preferred_element_type=jnp.float32) + bpi_ref[...]
    pi_ref[...] = jax.nn.sigmoid(logit)
    mu_ref[...] = jnp.dot(hn, wmu_ref[...], preferred_element_type=jnp.float32) + bmu_ref[...]


def _mlp(x_num, embs, w1a, w1bs, b1, g1, be1, w2, b2, g2, be2, wpi, bpi, wmu,
         bmu):
  def blk(p, i):
    return (jnp.where(p == 0, i, 0), 0)

  def const(p, i):
    return (0, 0)

  def out_blk(p, i):
    return (i, 0)

  grid = (3, NBLK)
  return pl.pallas_call(
      _mlp_body,
      grid=grid,
      in_specs=[pl.BlockSpec((RB, NUM_DIM), blk)]
      + [pl.BlockSpec((RB, 128), blk)] * NT
      + [pl.BlockSpec((NUM_DIM, H1), const)]
      + [pl.BlockSpec((128, H1), const)] * NT
      + [
          pl.BlockSpec((1, H1), const),
          pl.BlockSpec((1, H1), const),
          pl.BlockSpec((1, H1), const),
          pl.BlockSpec((H1, H2), const),
          pl.BlockSpec((1, H2), const),
          pl.BlockSpec((1, H2), const),
          pl.BlockSpec((1, H2), const),
          pl.BlockSpec((H2, 1), const),
          pl.BlockSpec((1, 1), const),
          pl.BlockSpec((H2, 1), const),
          pl.BlockSpec((1, 1), const),
      ],
      out_specs=[
          pl.BlockSpec((RB, 1), out_blk),
          pl.BlockSpec((RB, 1), out_blk),
      ],
      out_shape=[
          jax.ShapeDtypeStruct((B, 1), jnp.float32),
          jax.ShapeDtypeStruct((B, 1), jnp.float32),
      ],
      scratch_shapes=[
          pltpu.VMEM((B, H1), jnp.float32),
          pltpu.VMEM((B, H2), jnp.float32),
          pltpu.VMEM((1, H1), jnp.float32),
          pltpu.VMEM((1, H1), jnp.float32),
          pltpu.VMEM((1, H2), jnp.float32),
          pltpu.VMEM((1, H2), jnp.float32),
      ],
      compiler_params=pltpu.CompilerParams(
          dimension_semantics=("arbitrary", "arbitrary"),
          vmem_limit_bytes=100 * 1024 * 1024,
      ),
  )(x_num, *embs, w1a, *w1bs, b1, g1, be1, w2, b2, g2, be2, wpi, bpi, wmu, bmu)


def kernel(x_num, x_cat, tables, W1, b1, g1, be1, W2, b2, g2, be2, Wpi, bpi,
           Wmu, bmu):
  offs = jnp.concatenate([
      jnp.arange(NUM_FIELDS, dtype=jnp.int32) * VOCAB_P,
      jnp.zeros((NT * FPT - NUM_FIELDS,), jnp.int32),
  ])
  xpad = jnp.pad(x_cat, ((0, 0), (0, NT * FPT - NUM_FIELDS)))
  idx_t = (xpad + offs[None, :]).reshape(32, 4, 128, NT, FPT).transpose(
      0, 3, 1, 4, 2).reshape(IDX_ROWS, 128)

  tt = jnp.transpose(tables, (0, 2, 1))
  tails = tables[:, VTILES * 128:, :]
  aux = jnp.zeros((NUM_FIELDS, 8, 128), jnp.float32).at[:, :4].set(
      tails.reshape(NUM_FIELDS, 4, 128)).reshape(NUM_FIELDS * 1024)
  tab_lin = _sc_transpose(tt, aux).reshape(NUM_FIELDS * VOCAB_P, EMB_DIM)
  embs = _sc_gather(tab_lin, idx_t)

  w1a = W1[:NUM_DIM]
  w1b = jnp.zeros((NT * FPT * EMB_DIM, H1), W1.dtype).at[:NUM_FIELDS * EMB_DIM].set(
      W1[NUM_DIM:])
  w1bs = [w1b[t * 128:(t + 1) * 128] for t in range(NT)]
  pi, mu = _mlp(x_num, embs, w1a, w1bs, b1.reshape(1, H1), g1.reshape(1, H1),
                be1.reshape(1, H1), W2, b2.reshape(1, H2), g2.reshape(1, H2),
                be2.reshape(1, H2), Wpi, bpi.reshape(1, 1), Wmu,
                bmu.reshape(1, 1))
  return (pi, mu)


# speedup vs baseline: 1.0148x; 1.0148x over previous
import functools

import jax
import jax.numpy as jnp
from jax import lax
from jax.experimental import pallas as pl
from jax.experimental.pallas import tpu as pltpu
from jax.experimental.pallas import tpu_sc as plsc

B = 16384
NUM_DIM = 13
NUM_FIELDS = 26
VOCAB = 100000
EMB_DIM = 16
EPS = 1e-5

NT = 4
FPT = 8
IDX_ROWS = NT * B * FPT // 128

VTILES = VOCAB // 128
VTAIL = VOCAB - VTILES * 128
VOCAB_P = (VTILES + 1) * 128
RPF = VOCAB_P * EMB_DIM // 128
GRP = 11
NGRP = VTILES // GRP
ITEMS = NUM_FIELDS * NGRP

RB = 2048
NBLK = B // RB
H1 = 256
H2 = 128


def _sc_transpose(tt, aux):
  info = plsc.get_sparse_core_info()
  nw = info.num_cores * info.num_subcores
  n_iter = (ITEMS + nw - 1) // nw
  n_rounds = (n_iter + 1) // 2
  cols = GRP * 128
  orows = cols * EMB_DIM // 128
  mesh = plsc.VectorSubcoreMesh(core_axis_name="c", subcore_axis_name="s")

  @functools.partial(
      pl.kernel,
      mesh=mesh,
      compiler_params=pltpu.CompilerParams(use_tc_tiling_on_sc=True,
                                           needs_layout_passes=False),
      out_type=jax.ShapeDtypeStruct((NUM_FIELDS * RPF, 128), jnp.float32),
      scratch_types=[
          pltpu.VMEM((EMB_DIM, cols), jnp.float32),
          pltpu.VMEM((EMB_DIM, cols), jnp.float32),
          pltpu.VMEM((orows, 128), jnp.float32),
          pltpu.VMEM((orows, 128), jnp.float32),
          pltpu.SemaphoreType.DMA,
          pltpu.SemaphoreType.DMA,
          pltpu.SemaphoreType.DMA,
          pltpu.SemaphoreType.DMA,
      ],
  )
  def transpose_kernel(tt_hbm, aux_hbm, out_hbm, in0, in1, ob0, ob1, ws0, ws1,
                       is0, is1):
    wid = lax.axis_index("s") * info.num_cores + lax.axis_index("c")
    lanes = jax.lax.iota(jnp.int32, 16)
    ins = [in0, in1]
    obs = [ob0, ob1]
    wss = [ws0, ws1]
    iss = [is0, is1]

    pltpu.async_copy(tt_hbm.at[wid // NGRP, :,
                               pl.ds((wid - (wid // NGRP) * NGRP) * cols, cols)],
                     in0, is0)

    def round_body(r, carry):
      for b in range(2):
        i = 2 * r + b
        k = i * nw + wid

        @pl.when(k < ITEMS)
        def _(b=b, k=k):
          in_b, out_b, ws_b = ins[b], obs[b], wss[b]
          f = k // NGRP
          g = k - f * NGRP
          pltpu.make_async_copy(tt_hbm.at[0, :, pl.ds(0, cols)], in_b,
                                iss[b]).wait()
          kn = k + nw

          @pl.when(kn < ITEMS)
          def _prefetch():
            fn = kn // NGRP
            gn = kn - fn * NGRP
            pltpu.async_copy(tt_hbm.at[fn, :, pl.ds(gn * cols, cols)],
                             ins[(b + 1) % 2], iss[(b + 1) % 2])

          @pl.when(r > 0)
          def _drain():
            pltpu.make_async_copy(out_b, out_hbm.at[pl.ds(0, orows)],
                                  ws_b).wait()

          def tile_body(t, carry2):
            for j in range(8):
              cbase = t * 128 + j * 16
              wbase = cbase * 16
              for s in range(16):
                perm = (lanes + s) & 15
                vals = plsc.load_gather(in_b, [lanes, cbase + perm])
                w = wbase + perm * 16 + lanes
                plsc.store_scatter(out_b, [w >> 7, w & 127], vals)
            return carry2

          lax.fori_loop(0, GRP, tile_body, 0)
          pltpu.async_copy(out_b,
                           out_hbm.at[pl.ds(f * RPF + g * orows, orows)], ws_b)

      return carry

    lax.fori_loop(0, n_rounds, round_body, 0)
    for b in range(2):
      pltpu.make_async_copy(obs[b], out_hbm.at[pl.ds(0, orows)], wss[b]).wait()

    @pl.when(wid < NUM_FIELDS)
    def _tail():
      pltpu.sync_copy(aux_hbm.at[pl.ds(wid * 8, 8)], in0.at[:8, pl.ds(0, 128)])
      pltpu.sync_copy(in0.at[:8, pl.ds(0, 128)],
                      out_hbm.at[pl.ds(wid * RPF + VTILES * 16, 8)])

  return transpose_kernel(tt, aux)


def _sc_gather(tab_lin, idx_t):
  info = plsc.get_sparse_core_info()
  nw = info.num_cores * info.num_subcores
  bpw = B // nw
  n_chunks = NT * 4
  bpc = 128
  mesh = plsc.VectorSubcoreMesh(core_axis_name="c", subcore_axis_name="s")
  eshape = jax.ShapeDtypeStruct((B, 128), jnp.float32)

  @functools.partial(
      pl.kernel,
      mesh=mesh,
      compiler_params=pltpu.CompilerParams(use_tc_tiling_on_sc=False),
      out_type=[eshape] * NT,
      scratch_types=[
          pltpu.VMEM((128, 128), jnp.int32),
          pltpu.VMEM((FPT * bpc, EMB_DIM), jnp.float32),
          pltpu.VMEM((FPT * bpc, EMB_DIM), jnp.float32),
          pltpu.VMEM((bpc, 128), jnp.float32),
          pltpu.VMEM((bpc, 128), jnp.float32),
          pltpu.SemaphoreType.DMA,
          pltpu.SemaphoreType.DMA,
          pltpu.SemaphoreType.DMA,
          pltpu.SemaphoreType.DMA,
      ],
  )
  def gather_kernel(tab_hbm, idx_hbm, e0, e1, e2, e3, idx_v, ga, gb, ta, tb,
                    gsem_a, gsem_b, wsem_a, wsem_b):
    wid = lax.axis_index("s") * info.num_cores + lax.axis_index("c")
    outs = [e0, e1, e2, e3]
    gbufs = [ga, gb]
    tbufs = [ta, tb]
    gsems = [gsem_a, gsem_b]
    wsems = [wsem_a, wsem_b]

    pltpu.sync_copy(idx_hbm.at[pl.ds(wid * 128, 128)], idx_v)
    zeros16 = jnp.zeros((16,), jnp.float32)

    def live_fields(k):
      return FPT if k // 4 < NT - 1 else NUM_FIELDS - (NT - 1) * FPT

    def fire(k):
      gbuf = gbufs[k % 2]
      gsem = gsems[k % 2]
      return [
          pltpu.async_copy(tab_hbm.at[idx_v.at[k * FPT + j]],
                           gbuf.at[pl.ds(j * bpc, bpc)], gsem)
          for j in range(live_fields(k))
      ]

    pend_g = fire(0)
    pend_w = {}
    for k in range(n_chunks):
      for h in pend_g:
        h.wait()
      if k + 1 < n_chunks:
        pend_g = fire(k + 1)
      if k - 2 in pend_w:
        for h in pend_w.pop(k - 2):
          h.wait()
      gbuf = gbufs[k % 2]
      tbuf = tbufs[k % 2]
      nlive = live_fields(k)

      def repack(db, carry, gbuf=gbuf, tbuf=tbuf, nlive=nlive):
        for v in range(nlive):
          tbuf[db, pl.ds(v * EMB_DIM, EMB_DIM)] = gbuf[v * bpc + db, :]
        for v in range(nlive, FPT):
          tbuf[db, pl.ds(v * EMB_DIM, EMB_DIM)] = zeros16
        return carry

      lax.fori_loop(0, bpc, repack, 0)
      t = k // 4
      c = k - t * 4
      pend_w[k] = [
          pltpu.async_copy(tbuf, outs[t].at[pl.ds(wid * bpw + c * bpc, bpc)],
                           wsems[k % 2])
      ]
    for kk in sorted(pend_w):
      for h in pend_w[kk]:
        h.wait()

  return gather_kernel(tab_lin, idx_t)


def _mlp_body(xn_ref, e0_ref, e1_ref, e2_ref, e3_ref, w1a_ref, w1b0_ref,
              w1b1_ref, w1b2_ref, w1b3_ref, b1_ref, g1_ref, be1_ref, w2_ref,
              b2_ref, g2_ref, be2_ref, wpi_ref, bpi_ref, wmu_ref, bmu_ref,
              pi_ref, mu_ref, h1_s, h2_s, s1, q1, s2, q2):
  p = pl.program_id(0)
  i = pl.program_id(1)
  inv_b = 1.0 / B

  @pl.when(p == 0)
  def _phase0():
    @pl.when(i == 0)
    def _():
      s1[...] = jnp.zeros_like(s1)
      q1[...] = jnp.zeros_like(q1)

    h = (jnp.dot(xn_ref[...], w1a_ref[...], preferred_element_type=jnp.float32)
         + jnp.dot(e0_ref[...], w1b0_ref[...], preferred_element_type=jnp.float32)
         + jnp.dot(e1_ref[...], w1b1_ref[...], preferred_element_type=jnp.float32)
         + jnp.dot(e2_ref[...], w1b2_ref[...], preferred_element_type=jnp.float32)
         + jnp.dot(e3_ref[...], w1b3_ref[...], preferred_element_type=jnp.float32)
         + b1_ref[...])
    h1_s[pl.ds(i * RB, RB), :] = h
    s1[...] += jnp.sum(h, axis=0, keepdims=True)
    q1[...] += jnp.sum(h * h, axis=0, keepdims=True)

  @pl.when(p == 1)
  def _phase1():
    @pl.when(i == 0)
    def _():
      s2[...] = jnp.zeros_like(s2)
      q2[...] = jnp.zeros_like(q2)

    m = s1[...] * inv_b
    v = q1[...] * inv_b - m * m
    a = g1_ref[...] * lax.rsqrt(v + EPS)
    c = be1_ref[...] - m * a
    h = h1_s[pl.ds(i * RB, RB), :]
    hn = jnp.maximum(h * a + c, 0.0)
    h2 = jnp.dot(hn, w2_ref[...], preferred_element_type=jnp.float32) + b2_ref[...]
    h2_s[pl.ds(i * RB, RB), :] = h2
    s2[...] += jnp.sum(h2, axis=0, keepdims=True)
    q2[...] += jnp.sum(h2 * h2, axis=0, keepdims=True)

  @pl.when(p == 2)
  def _phase2():
    m = s2[...] * inv_b
    v = q2[...] * inv_b - m * m
    a = g2_ref[...] * lax.rsqrt(v + EPS)
    c = be2_ref[...] - m * a
    h = h2_s[pl.ds(i * RB, RB), :]
    hn = jnp.maximum(h * a + c, 0.0)
    logit = jnp.dot(hn, wpi_ref[...], preferred_element_type=jnp.float32) + bpi_ref[...]
    pi_ref[...] = jax.nn.sigmoid(logit)
    mu_ref[...] = jnp.dot(hn, wmu_ref[...], preferred_element_type=jnp.float32) + bmu_ref[...]


def _mlp(x_num, embs, w1a, w1bs, b1, g1, be1, w2, b2, g2, be2, wpi, bpi, wmu,
         bmu):
  def blk(p, i):
    return (jnp.where(p == 0, i, 0), 0)

  def const(p, i):
    return (0, 0)

  def out_blk(p, i):
    return (i, 0)

  grid = (3, NBLK)
  return pl.pallas_call(
      _mlp_body,
      grid=grid,
      in_specs=[pl.BlockSpec((RB, NUM_DIM), blk)]
      + [pl.BlockSpec((RB, 128), blk)] * NT
      + [pl.BlockSpec((NUM_DIM, H1), const)]
      + [pl.BlockSpec((128, H1), const)] * NT
      + [
          pl.BlockSpec((1, H1), const),
          pl.BlockSpec((1, H1), const),
          pl.BlockSpec((1, H1), const),
          pl.BlockSpec((H1, H2), const),
          pl.BlockSpec((1, H2), const),
          pl.BlockSpec((1, H2), const),
          pl.BlockSpec((1, H2), const),
          pl.BlockSpec((H2, 1), const),
          pl.BlockSpec((1, 1), const),
          pl.BlockSpec((H2, 1), const),
          pl.BlockSpec((1, 1), const),
      ],
      out_specs=[
          pl.BlockSpec((RB, 1), out_blk),
          pl.BlockSpec((RB, 1), out_blk),
      ],
      out_shape=[
          jax.ShapeDtypeStruct((B, 1), jnp.float32),
          jax.ShapeDtypeStruct((B, 1), jnp.float32),
      ],
      scratch_shapes=[
          pltpu.VMEM((B, H1), jnp.float32),
          pltpu.VMEM((B, H2), jnp.float32),
          pltpu.VMEM((1, H1), jnp.float32),
          pltpu.VMEM((1, H1), jnp.float32),
          pltpu.VMEM((1, H2), jnp.float32),
          pltpu.VMEM((1, H2), jnp.float32),
      ],
      compiler_params=pltpu.CompilerParams(
          dimension_semantics=("arbitrary", "arbitrary"),
          vmem_limit_bytes=100 * 1024 * 1024,
      ),
  )(x_num, *embs, w1a, *w1bs, b1, g1, be1, w2, b2, g2, be2, wpi, bpi, wmu, bmu)


def kernel(x_num, x_cat, tables, W1, b1, g1, be1, W2, b2, g2, be2, Wpi, bpi,
           Wmu, bmu):
  offs = jnp.concatenate([
      jnp.arange(NUM_FIELDS, dtype=jnp.int32) * VOCAB_P,
      jnp.zeros((NT * FPT - NUM_FIELDS,), jnp.int32),
  ])
  xpad = jnp.pad(x_cat, ((0, 0), (0, NT * FPT - NUM_FIELDS)))
  idx_t = (xpad + offs[None, :]).reshape(32, 4, 128, NT, FPT).transpose(
      0, 3, 1, 4, 2).reshape(IDX_ROWS, 128)

  tt = jnp.transpose(tables, (0, 2, 1))
  tails = tables[:, VTILES * 128:, :]
  aux = jnp.zeros((NUM_FIELDS, 8, 128), jnp.float32).at[:, :4].set(
      tails.reshape(NUM_FIELDS, 4, 128)).reshape(NUM_FIELDS * 8, 128)
  tab_lin = _sc_transpose(tt, aux).reshape(NUM_FIELDS * VOCAB_P, EMB_DIM)
  embs = _sc_gather(tab_lin, idx_t)

  w1a = W1[:NUM_DIM]
  w1b = jnp.zeros((NT * FPT * EMB_DIM, H1), W1.dtype).at[:NUM_FIELDS * EMB_DIM].set(
      W1[NUM_DIM:])
  w1bs = [w1b[t * 128:(t + 1) * 128] for t in range(NT)]
  pi, mu = _mlp(x_num, embs, w1a, w1bs, b1.reshape(1, H1), g1.reshape(1, H1),
                be1.reshape(1, H1), W2, b2.reshape(1, H2), g2.reshape(1, H2),
                be2.reshape(1, H2), Wpi, bpi.reshape(1, 1), Wmu,
                bmu.reshape(1, 1))
  return (pi, mu)
